# baseline (speedup 1.0000x reference)
import functools

import jax
import jax.numpy as jnp
from jax import lax
from jax.experimental import pallas as pl
from jax.experimental.pallas import tpu as pltpu
from jax.experimental.pallas import tpu_sc as plsc

N_SEQ, MAX_LEN, H_DIM, N_CHAR = 16, 4096, 128, 256
N_CHARS_TOT = 32768
N_GROUPS, GSIZE = 8, 4096

NROW = N_SEQ * MAX_LEN
D = H_DIM
NC, NS, L = 2, 16, 16
NW = NC * NS
RPW = NROW // NW
DUMMY_ROW = NROW
NROWH = NROW + 128

CSLAB = 8192
CPASS = (NROW // NC) // CSLAB
CCHUNK = N_CHARS_TOT // NS
SLAB_DUMMY = CSLAB

BS = 128

_i32 = jnp.int32
_f32 = jnp.float32


def _mesh():
  return plsc.VectorSubcoreMesh(core_axis_name="c", subcore_axis_name="s",
                                num_cores=NC, num_subcores=NS)


_SC_PARAMS = pltpu.CompilerParams(needs_layout_passes=False)


def _igather(src_hbm, idx_ref, dst_vmem, sem):
  pltpu.async_copy(src_hbm.at[idx_ref], dst_vmem, sem).wait()


def _iscatter(src_vmem, idx_ref, dst_hbm, sem):
  pltpu.async_copy(src_vmem, dst_hbm.at[idx_ref], sem).wait()


def _iscatter_add(src_vmem, idx_ref, dst_spmem):
  pltpu.sync_copy(src_vmem, dst_spmem.at[idx_ref], add=True)


def _compact(m, trash, cur):
  mi = m.astype(_i32)
  offs = cur + plsc.cumsum(mi) - mi
  dst = jnp.where(m, offs, trash)
  return dst, cur + jnp.sum(mi)


def _char_body(seq_h, pos_h, ids_h, emb_h, h_ref,
               seq_v, pos_v, ids_v, sel_l, sel_i,
               zbuf, rows_v, slab, sem):
  c = lax.axis_index("c")
  s = lax.axis_index("s")
  cslab_pt = CSLAB // NS
  zr = min(64, cslab_pt)

  @pl.loop(0, zr)
  def _(i):
    for j in range(D // L):
      zbuf[i, pl.ds(j * L, L)] = jnp.zeros((L,), _f32)

  ub = s * CCHUNK
  pltpu.sync_copy(seq_h.at[pl.ds(ub, CCHUNK)], seq_v)
  pltpu.sync_copy(pos_h.at[pl.ds(ub, CCHUNK)], pos_v)
  pltpu.sync_copy(ids_h.at[pl.ds(ub, CCHUNK)], ids_v)

  @pl.loop(0, CCHUNK // L)
  def _(i):
    sl = pl.ds(i * L, L)
    seq_v[sl] = seq_v[sl] * MAX_LEN + pos_v[sl]

  for p in range(CPASS):
    lo = c * (NROW // NC) + p * CSLAB

    for k in range(cslab_pt // zr):
      pltpu.sync_copy(zbuf, slab.at[pl.ds(s * cslab_pt + k * zr, zr)])
    plsc.subcore_barrier()

    @pl.loop(0, CCHUNK // BS + 1)
    def _(r):
      for j in range(BS // L):
        sl = pl.ds(j * L, L)
        col = lax.iota(_i32, L) + j * L
        sel_l[r, sl] = col + SLAB_DUMMY
        sel_i[r, sl] = col

    def cbody(i, cur):
      sl = pl.ds(i * L, L)
      lv = seq_v[sl]
      m = (lv >= lo) & (lv < lo + CSLAB)
      dst, nxt = _compact(m, CCHUNK, cur)
      dr = dst // BS
      dc = dst % BS
      plsc.store_scatter(sel_l, [dr, dc], lv - lo)
      plsc.store_scatter(sel_i, [dr, dc], ids_v[sl])
      return nxt
    n = lax.fori_loop(0, CCHUNK // L, cbody, 0)

    def bbody(b, _):
      _igather(emb_h, sel_i.at[b], rows_v, sem)
      _iscatter_add(rows_v, sel_l.at[b], slab)
      return 0
    lax.fori_loop(0, (n + BS - 1) // BS, bbody, 0)

    plsc.subcore_barrier()

    cl = s * cslab_pt
    pltpu.sync_copy(slab.at[pl.ds(cl, cslab_pt)],
                    h_ref.at[pl.ds(lo + cl, cslab_pt)])


def _char_kernel():
  return pl.kernel(
      _char_body,
      out_type=(),
      mesh=_mesh(),
      scratch_types=[
          pltpu.VMEM((CCHUNK,), _i32),
          pltpu.VMEM((CCHUNK,), _i32),
          pltpu.VMEM((CCHUNK,), _i32),
          pltpu.VMEM((CCHUNK // BS + 1, BS), _i32),
          pltpu.VMEM((CCHUNK // BS + 1, BS), _i32),
          pltpu.VMEM((min(64, CSLAB // NS), D), _f32),
          pltpu.VMEM((BS, D), _f32),
          pltpu.VMEM_SHARED((CSLAB + BS, D), _f32),
          pltpu.SemaphoreType.DMA,
      ],
      name="lzd_char_scatter",
      compiler_params=_SC_PARAMS,
  )


def _gather_body(h_ref, seq_h, fir_h, sec_h, a_out, b_out,
                 sq, f1, f2, lin1, lin2, bufa, bufb, sema, semb):
  c = lax.axis_index("c")
  s = lax.axis_index("s")
  wid = s * NC + c
  base = wid * (GSIZE // NW)
  nrows = GSIZE // NW

  pltpu.sync_copy(seq_h.at[pl.ds(base, nrows)], sq)
  pltpu.sync_copy(fir_h.at[pl.ds(base, nrows)], f1)
  pltpu.sync_copy(sec_h.at[pl.ds(base, nrows)], f2)

  @pl.loop(0, nrows // L)
  def _(i):
    sl = pl.ds(i * L, L)
    sv = sq[sl] * MAX_LEN
    lin1[sl] = sv + f1[sl]
    lin2[sl] = sv + f2[sl]

  d1 = pltpu.async_copy(h_ref.at[lin1], bufa, sema)
  d2 = pltpu.async_copy(h_ref.at[lin2], bufb, semb)
  d1.wait()
  d2.wait()
  pltpu.sync_copy(bufa, a_out.at[pl.ds(base, nrows)])
  pltpu.sync_copy(bufb, b_out.at[pl.ds(base, nrows)])


def _gather_kernel():
  nrows = GSIZE // NW
  return pl.kernel(
      _gather_body,
      out_type=(jax.ShapeDtypeStruct((GSIZE, D), _f32),
                jax.ShapeDtypeStruct((GSIZE, D), _f32)),
      mesh=_mesh(),
      scratch_types=[
          pltpu.VMEM((nrows,), _i32),
          pltpu.VMEM((nrows,), _i32),
          pltpu.VMEM((nrows,), _i32),
          pltpu.VMEM((nrows,), _i32),
          pltpu.VMEM((nrows,), _i32),
          pltpu.VMEM((nrows, D), _f32),
          pltpu.VMEM((nrows, D), _f32),
          pltpu.SemaphoreType.DMA,
          pltpu.SemaphoreType.DMA,
      ],
      name="lzd_gather",
      compiler_params=_SC_PARAMS,
  )


def _comp_body(a_ref, b_ref, w1_ref, w2_ref, bias_ref, o_ref):
  acc = jnp.dot(a_ref[...], w1_ref[...], preferred_element_type=_f32)
  acc += jnp.dot(b_ref[...], w2_ref[...], preferred_element_type=_f32)
  o_ref[...] = jnp.tanh(acc + bias_ref[...])


def _comp_kernel():
  blk = 512
  return pl.pallas_call(
      _comp_body,
      grid=(GSIZE // blk,),
      in_specs=[
          pl.BlockSpec((blk, D), lambda i: (i, 0)),
          pl.BlockSpec((blk, D), lambda i: (i, 0)),
          pl.BlockSpec((D, D), lambda i: (0, 0)),
          pl.BlockSpec((D, D), lambda i: (0, 0)),
          pl.BlockSpec((1, D), lambda i: (0, 0)),
      ],
      out_specs=pl.BlockSpec((blk, D), lambda i: (i, 0)),
      out_shape=jax.ShapeDtypeStruct((GSIZE, D), _f32),
  )


GCH = GSIZE // NS


def _scatter_body(last, h_ref, comp_h, seq_h, pos_h,
                  lv_v, ps_v, sel_l, sel_i, hidx, hrows, crows, slab, zb,
                  sem, semd, semw):
  c = lax.axis_index("c")
  s = lax.axis_index("s")
  iota = lax.iota(_i32, L)
  ub = s * GCH

  pltpu.sync_copy(seq_h.at[pl.ds(ub, GCH)], lv_v)
  pltpu.sync_copy(pos_h.at[pl.ds(ub, GCH)], ps_v)
  @pl.loop(0, GCH // L)
  def _(i):
    sl = pl.ds(i * L, L)
    lv_v[sl] = lv_v[sl] * MAX_LEN + ps_v[sl]

  if last:
    @pl.loop(0, D // L)
    def _(j):
      zb[0, pl.ds(j * L, L)] = jnp.zeros((L,), _f32)

  for p in range(CPASS):
    lo = c * (NROW // NC) + p * CSLAB

    @pl.loop(0, GCH // BS)
    def _(r):
      for j in range(BS // L):
        sl = pl.ds(j * L, L)
        col = lax.iota(_i32, L) + j * L
        sel_l[r, sl] = col + SLAB_DUMMY
        sel_i[r, sl] = col

    def cbody(i, cur):
      sl = pl.ds(i * L, L)
      lvv = lv_v[sl]
      m = (lvv >= lo) & (lvv < lo + CSLAB)
      dst, nxt = _compact(m, GCH, cur)
      dr = dst // BS
      dc = dst % BS
      plsc.store_scatter(sel_l, [dr, dc], lvv - lo)
      plsc.store_scatter(sel_i, [dr, dc], iota + ub + i * L)
      return nxt
    n = lax.fori_loop(0, GCH // L, cbody, 0)
    nb = (n + BS - 1) // BS

    def lbody(b, _):
      @pl.loop(0, BS // L)
      def _(j):
        sl = pl.ds(j * L, L)
        slot = sel_l[b, sl]
        hidx[sl] = jnp.where(slot >= SLAB_DUMMY,
                             DUMMY_ROW + slot - SLAB_DUMMY, lo + slot)
      _igather(h_ref, hidx, hrows, semd)
      pltpu.sync_copy(hrows, slab.at[sel_l.at[b]])
      return 0
    lax.fori_loop(0, nb, lbody, 0)
    plsc.subcore_barrier()

    def abody(b, _):
      _igather(comp_h, sel_i.at[b], crows, sem)
      _iscatter_add(crows, sel_l.at[b], slab)
      return 0
    lax.fori_loop(0, nb, abody, 0)
    plsc.subcore_barrier()

    def wbody(b, _):
      @pl.loop(0, BS // L)
      def _(j):
        sl = pl.ds(j * L, L)
        slot = sel_l[b, sl]
        hidx[sl] = jnp.where(slot >= SLAB_DUMMY,
                             DUMMY_ROW + slot - SLAB_DUMMY, lo + slot)
      pltpu.sync_copy(slab.at[sel_l.at[b]], hrows)
      _iscatter(hrows, hidx, h_ref, semw)
      return 0
    lax.fori_loop(0, nb, wbody, 0)
    plsc.subcore_barrier()

  if last:
    @pl.when(s < N_SEQ // NC)
    def _():
      pltpu.sync_copy(zb, h_ref.at[pl.ds(c * (NROW // NC) + s * MAX_LEN, 1)])


def _scatter_kernel(last):
  return pl.kernel(
      functools.partial(_scatter_body, last),
      out_type=(),
      mesh=_mesh(),
      scratch_types=[
          pltpu.VMEM((GCH,), _i32),
          pltpu.VMEM((GCH,), _i32),
          pltpu.VMEM((GCH // BS + 1, BS), _i32),
          pltpu.VMEM((GCH // BS + 1, BS), _i32),
          pltpu.VMEM((BS,), _i32),
          pltpu.VMEM((BS, D), _f32),
          pltpu.VMEM((BS, D), _f32),
          pltpu.VMEM_SHARED((CSLAB + BS, D), _f32),
          pltpu.VMEM((1, D), _f32),
          pltpu.SemaphoreType.DMA,
          pltpu.SemaphoreType.DMA,
          pltpu.SemaphoreType.DMA,
      ],
      name="lzd_scatter_add",
      compiler_params=_SC_PARAMS,
  )


def kernel(char_i_seq, char_i_pos, char_ids, group_i_seq, group_i_first,
           group_i_second, group_i_pos, emb_table, W1, W2, bias):
  h_ref = jax.new_ref(jnp.zeros((NROWH, D), _f32))

  _char_kernel()(char_i_seq, char_i_pos, char_ids, emb_table, h_ref)
  comp_fn = _comp_kernel()
  gather_fn = _gather_kernel()
  bias2d = bias.reshape(1, D)
  for g in range(N_GROUPS):
    a_mat, b_mat = gather_fn(h_ref, group_i_seq[g], group_i_first[g],
                             group_i_second[g])
    comp = comp_fn(a_mat, b_mat, W1, W2, bias2d)
    _scatter_kernel(g == N_GROUPS - 1)(h_ref, comp, group_i_seq[g],
                                       group_i_pos[g])

  return h_ref[...][:NROW].reshape(N_SEQ, MAX_LEN, D)


# speedup vs baseline: 1.1869x; 1.1869x over previous
import functools

import jax
import jax.numpy as jnp
from jax import lax
from jax.experimental import pallas as pl
from jax.experimental.pallas import tpu as pltpu
from jax.experimental.pallas import tpu_sc as plsc

N_SEQ, MAX_LEN, H_DIM, N_CHAR = 16, 4096, 128, 256
N_CHARS_TOT = 32768
N_GROUPS, GSIZE = 8, 4096

NROW = N_SEQ * MAX_LEN
D = H_DIM
NC, NS, L = 2, 16, 16
NW = NC * NS
RPW = NROW // NW
DUMMY_ROW = NROW
NROWH = NROW + 128

CSLAB = 8192
CPASS = (NROW // NC) // CSLAB
CCHUNK = N_CHARS_TOT // NS
SLAB_DUMMY = CSLAB

BS = 128

_i32 = jnp.int32
_f32 = jnp.float32


def _mesh():
  return plsc.VectorSubcoreMesh(core_axis_name="c", subcore_axis_name="s",
                                num_cores=NC, num_subcores=NS)


_SC_PARAMS = pltpu.CompilerParams(needs_layout_passes=False)


def _igather(src_hbm, idx_ref, dst_vmem, sem):
  pltpu.async_copy(src_hbm.at[idx_ref], dst_vmem, sem).wait()


def _iscatter(src_vmem, idx_ref, dst_hbm, sem):
  pltpu.async_copy(src_vmem, dst_hbm.at[idx_ref], sem).wait()


def _iscatter_add(src_vmem, idx_ref, dst_spmem):
  pltpu.sync_copy(src_vmem, dst_spmem.at[idx_ref], add=True)


def _compact(m, trash, cur):
  mi = m.astype(_i32)
  offs = cur + plsc.cumsum(mi) - mi
  dst = jnp.where(m, offs, trash)
  return dst, cur + jnp.sum(mi)


def _char_body(seq_h, pos_h, ids_h, emb_h, h_ref,
               seq_v, pos_v, ids_v, sel_l, sel_i,
               zbuf, rows_v, slab, sem):
  c = lax.axis_index("c")
  s = lax.axis_index("s")
  cslab_pt = CSLAB // NS
  zr = min(64, cslab_pt)

  @pl.loop(0, zr)
  def _(i):
    for j in range(D // L):
      zbuf[i, pl.ds(j * L, L)] = jnp.zeros((L,), _f32)

  ub = s * CCHUNK
  pltpu.sync_copy(seq_h.at[pl.ds(ub, CCHUNK)], seq_v)
  pltpu.sync_copy(pos_h.at[pl.ds(ub, CCHUNK)], pos_v)
  pltpu.sync_copy(ids_h.at[pl.ds(ub, CCHUNK)], ids_v)

  @pl.loop(0, CCHUNK // L)
  def _(i):
    sl = pl.ds(i * L, L)
    seq_v[sl] = seq_v[sl] * MAX_LEN + pos_v[sl]

  for p in range(CPASS):
    lo = c * (NROW // NC) + p * CSLAB

    for k in range(cslab_pt // zr):
      pltpu.sync_copy(zbuf, slab.at[pl.ds(s * cslab_pt + k * zr, zr)])
    plsc.subcore_barrier()

    @pl.loop(0, CCHUNK // BS + 1)
    def _(r):
      for j in range(BS // L):
        sl = pl.ds(j * L, L)
        col = lax.iota(_i32, L) + j * L
        sel_l[r, sl] = col + SLAB_DUMMY
        sel_i[r, sl] = col

    def cbody(i, cur):
      sl = pl.ds(i * L, L)
      lv = seq_v[sl]
      m = (lv >= lo) & (lv < lo + CSLAB)
      dst, nxt = _compact(m, CCHUNK, cur)
      dr = dst // BS
      dc = dst % BS
      plsc.store_scatter(sel_l, [dr, dc], lv - lo)
      plsc.store_scatter(sel_i, [dr, dc], ids_v[sl])
      return nxt
    n = lax.fori_loop(0, CCHUNK // L, cbody, 0)

    def bbody(b, _):
      _igather(emb_h, sel_i.at[b], rows_v, sem)
      _iscatter_add(rows_v, sel_l.at[b], slab)
      return 0
    lax.fori_loop(0, (n + BS - 1) // BS, bbody, 0)

    plsc.subcore_barrier()

    cl = s * cslab_pt
    pltpu.sync_copy(slab.at[pl.ds(cl, cslab_pt)],
                    h_ref.at[pl.ds(lo + cl, cslab_pt)])


def _char_kernel():
  return pl.kernel(
      _char_body,
      out_type=(),
      mesh=_mesh(),
      scratch_types=[
          pltpu.VMEM((CCHUNK,), _i32),
          pltpu.VMEM((CCHUNK,), _i32),
          pltpu.VMEM((CCHUNK,), _i32),
          pltpu.VMEM((CCHUNK // BS + 1, BS), _i32),
          pltpu.VMEM((CCHUNK // BS + 1, BS), _i32),
          pltpu.VMEM((min(64, CSLAB // NS), D), _f32),
          pltpu.VMEM((BS, D), _f32),
          pltpu.VMEM_SHARED((CSLAB + BS, D), _f32),
          pltpu.SemaphoreType.DMA,
      ],
      name="lzd_char_scatter",
      compiler_params=_SC_PARAMS,
  )


def _gather_body(h_ref, seq_h, fir_h, sec_h, a_out, b_out,
                 sq, f1, f2, lin1, lin2, bufa, bufb, sema, semb):
  c = lax.axis_index("c")
  s = lax.axis_index("s")
  wid = s * NC + c
  base = wid * (GSIZE // NW)
  nrows = GSIZE // NW

  pltpu.sync_copy(seq_h.at[pl.ds(base, nrows)], sq)
  pltpu.sync_copy(fir_h.at[pl.ds(base, nrows)], f1)
  pltpu.sync_copy(sec_h.at[pl.ds(base, nrows)], f2)

  @pl.loop(0, nrows // L)
  def _(i):
    sl = pl.ds(i * L, L)
    sv = sq[sl] * MAX_LEN
    lin1[sl] = sv + f1[sl]
    lin2[sl] = sv + f2[sl]

  d1 = pltpu.async_copy(h_ref.at[lin1], bufa, sema)
  d2 = pltpu.async_copy(h_ref.at[lin2], bufb, semb)
  d1.wait()
  d2.wait()
  pltpu.sync_copy(bufa, a_out.at[pl.ds(base, nrows)])
  pltpu.sync_copy(bufb, b_out.at[pl.ds(base, nrows)])


def _gather_kernel():
  nrows = GSIZE // NW
  return pl.kernel(
      _gather_body,
      out_type=(jax.ShapeDtypeStruct((GSIZE, D), _f32),
                jax.ShapeDtypeStruct((GSIZE, D), _f32)),
      mesh=_mesh(),
      scratch_types=[
          pltpu.VMEM((nrows,), _i32),
          pltpu.VMEM((nrows,), _i32),
          pltpu.VMEM((nrows,), _i32),
          pltpu.VMEM((nrows,), _i32),
          pltpu.VMEM((nrows,), _i32),
          pltpu.VMEM((nrows, D), _f32),
          pltpu.VMEM((nrows, D), _f32),
          pltpu.SemaphoreType.DMA,
          pltpu.SemaphoreType.DMA,
      ],
      name="lzd_gather",
      compiler_params=_SC_PARAMS,
  )


def _comp_body(a_ref, b_ref, w1_ref, w2_ref, bias_ref, o_ref):
  acc = jnp.dot(a_ref[...], w1_ref[...], preferred_element_type=_f32)
  acc += jnp.dot(b_ref[...], w2_ref[...], preferred_element_type=_f32)
  o_ref[...] = jnp.tanh(acc + bias_ref[...])


def _comp_kernel():
  blk = 512
  return pl.pallas_call(
      _comp_body,
      grid=(GSIZE // blk,),
      in_specs=[
          pl.BlockSpec((blk, D), lambda i: (i, 0)),
          pl.BlockSpec((blk, D), lambda i: (i, 0)),
          pl.BlockSpec((D, D), lambda i: (0, 0)),
          pl.BlockSpec((D, D), lambda i: (0, 0)),
          pl.BlockSpec((1, D), lambda i: (0, 0)),
      ],
      out_specs=pl.BlockSpec((blk, D), lambda i: (i, 0)),
      out_shape=jax.ShapeDtypeStruct((GSIZE, D), _f32),
  )


GCH = GSIZE // NS
SSLAB = 10944
SPASSES = ((0, SSLAB), (SSLAB, SSLAB), (2 * SSLAB, NROW // NC - 2 * SSLAB))


def _scatter_body(last, h_ref, comp_h, seq_h, pos_h,
                  lv_v, ps_v, sel_l, sel_i, hidx, hrows, crows, slab, zb,
                  sem, semd, semw):
  c = lax.axis_index("c")
  s = lax.axis_index("s")
  iota = lax.iota(_i32, L)
  ub = s * GCH

  pltpu.sync_copy(seq_h.at[pl.ds(ub, GCH)], lv_v)
  pltpu.sync_copy(pos_h.at[pl.ds(ub, GCH)], ps_v)
  @pl.loop(0, GCH // L)
  def _(i):
    sl = pl.ds(i * L, L)
    lv_v[sl] = lv_v[sl] * MAX_LEN + ps_v[sl]

  if last:
    @pl.loop(0, D // L)
    def _(j):
      zb[0, pl.ds(j * L, L)] = jnp.zeros((L,), _f32)

  for off, sz in SPASSES:
    lo = c * (NROW // NC) + off

    @pl.loop(0, GCH // BS)
    def _(r):
      for j in range(BS // L):
        sl = pl.ds(j * L, L)
        col = lax.iota(_i32, L) + j * L
        sel_l[r, sl] = col + SSLAB
        sel_i[r, sl] = col

    def cbody(i, cur):
      sl = pl.ds(i * L, L)
      lvv = lv_v[sl]
      m = (lvv >= lo) & (lvv < lo + sz)
      dst, nxt = _compact(m, GCH, cur)
      dr = dst // BS
      dc = dst % BS
      plsc.store_scatter(sel_l, [dr, dc], lvv - lo)
      plsc.store_scatter(sel_i, [dr, dc], iota + ub + i * L)
      return nxt
    n = lax.fori_loop(0, GCH // L, cbody, 0)
    nb = (n + BS - 1) // BS

    def lbody(b, _):
      @pl.loop(0, BS // L)
      def _(j):
        sl = pl.ds(j * L, L)
        slot = sel_l[b, sl]
        hidx[sl] = jnp.where(slot >= SSLAB,
                             DUMMY_ROW + slot - SSLAB, lo + slot)
      _igather(h_ref, hidx, hrows, semd)
      pltpu.sync_copy(hrows, slab.at[sel_l.at[b]])
      return 0
    lax.fori_loop(0, nb, lbody, 0)
    plsc.subcore_barrier()

    def abody(b, _):
      _igather(comp_h, sel_i.at[b], crows, sem)
      _iscatter_add(crows, sel_l.at[b], slab)
      return 0
    lax.fori_loop(0, nb, abody, 0)
    plsc.subcore_barrier()

    def wbody(b, _):
      @pl.loop(0, BS // L)
      def _(j):
        sl = pl.ds(j * L, L)
        slot = sel_l[b, sl]
        hidx[sl] = jnp.where(slot >= SSLAB,
                             DUMMY_ROW + slot - SSLAB, lo + slot)
      pltpu.sync_copy(slab.at[sel_l.at[b]], hrows)
      _iscatter(hrows, hidx, h_ref, semw)
      return 0
    lax.fori_loop(0, nb, wbody, 0)
    plsc.subcore_barrier()

  if last:
    @pl.when(s < N_SEQ // NC)
    def _():
      pltpu.sync_copy(zb, h_ref.at[pl.ds(c * (NROW // NC) + s * MAX_LEN, 1)])


def _scatter_kernel(last):
  return pl.kernel(
      functools.partial(_scatter_body, last),
      out_type=(),
      mesh=_mesh(),
      scratch_types=[
          pltpu.VMEM((GCH,), _i32),
          pltpu.VMEM((GCH,), _i32),
          pltpu.VMEM((GCH // BS + 1, BS), _i32),
          pltpu.VMEM((GCH // BS + 1, BS), _i32),
          pltpu.VMEM((BS,), _i32),
          pltpu.VMEM((BS, D), _f32),
          pltpu.VMEM((BS, D), _f32),
          pltpu.VMEM_SHARED((SSLAB + BS, D), _f32),
          pltpu.VMEM((1, D), _f32),
          pltpu.SemaphoreType.DMA,
          pltpu.SemaphoreType.DMA,
          pltpu.SemaphoreType.DMA,
      ],
      name="lzd_scatter_add",
      compiler_params=_SC_PARAMS,
  )


def kernel(char_i_seq, char_i_pos, char_ids, group_i_seq, group_i_first,
           group_i_second, group_i_pos, emb_table, W1, W2, bias):
  h_ref = jax.new_ref(jnp.zeros((NROWH, D), _f32))

  _char_kernel()(char_i_seq, char_i_pos, char_ids, emb_table, h_ref)
  comp_fn = _comp_kernel()
  gather_fn = _gather_kernel()
  bias2d = bias.reshape(1, D)
  for g in range(N_GROUPS):
    a_mat, b_mat = gather_fn(h_ref, group_i_seq[g], group_i_first[g],
                             group_i_second[g])
    comp = comp_fn(a_mat, b_mat, W1, W2, bias2d)
    _scatter_kernel(g == N_GROUPS - 1)(h_ref, comp, group_i_seq[g],
                                       group_i_pos[g])

  return h_ref[...][:NROW].reshape(N_SEQ, MAX_LEN, D)
